# Initial kernel scaffold; baseline (speedup 1.0000x reference)
import jax
import jax.numpy as jnp
from jax.experimental import pallas as pl


def kernel(feat_spec, state):
    raise NotImplementedError("write your pallas kernel here")


# speedup vs baseline: 8.7325x; 8.7325x over previous
import functools

import jax
import jax.numpy as jnp
import numpy as np
from jax.experimental import pallas as pl
from jax.experimental.pallas import tpu as pltpu

_ALPHA = 0.99
_L = 200


def _decay_matrix(l: int) -> np.ndarray:
    i = np.arange(l, dtype=np.float64)
    p = i[:, None] - i[None, :]
    mat = np.tril((1.0 - _ALPHA) * np.power(_ALPHA, np.maximum(p, 0.0)))
    return mat.astype(np.float32)


def _ema_body(a_mat_ref, state_ref, x_ref, out_ref, fs_ref, carry_ref, *,
              n_chunks: int, l: int):
    t = pl.program_id(1)

    @pl.when(t == 0)
    def _():
        carry_ref[...] = state_ref[0]

    x = x_ref[0]
    a = x[0] * x[0] + x[1] * x[1]

    s_prev = carry_ref[...]
    rows = jax.lax.broadcasted_iota(jnp.int32, a.shape, 0)
    inject = (_ALPHA / (1.0 - _ALPHA)) * s_prev
    a = a + jnp.where(rows == 0, jnp.broadcast_to(inject, a.shape), 0.0)

    s_all = jnp.dot(a_mat_ref[...], a,
                    preferred_element_type=jnp.float32,
                    precision=jax.lax.Precision.HIGHEST)

    inv = jax.lax.rsqrt(s_all)
    out_ref[0, 0] = x[0] * inv
    out_ref[0, 1] = x[1] * inv

    carry_ref[...] = s_all[l - 1:l, :]

    @pl.when(t == n_chunks - 1)
    def _():
        fs_ref[0] = s_all[l - 1:l, :]


def kernel(feat_spec, state):
    b, c, t, f = feat_spec.shape
    l = _L
    n_chunks = t // l
    a_mat = jnp.asarray(_decay_matrix(l))
    state_in = state.astype(feat_spec.dtype)

    body = functools.partial(_ema_body, n_chunks=n_chunks, l=l)
    out, final_state = pl.pallas_call(
        body,
        grid=(b, n_chunks),
        in_specs=[
            pl.BlockSpec((l, l), lambda i, j: (0, 0)),
            pl.BlockSpec((1, 1, f), lambda i, j: (0, 0, 0)),
            pl.BlockSpec((1, c, l, f), lambda i, j: (i, 0, j, 0)),
        ],
        out_specs=[
            pl.BlockSpec((1, c, l, f), lambda i, j: (i, 0, j, 0)),
            pl.BlockSpec((1, 1, f), lambda i, j: (i, 0, 0)),
        ],
        out_shape=[
            jax.ShapeDtypeStruct((b, c, t, f), feat_spec.dtype),
            jax.ShapeDtypeStruct((b, 1, f), feat_spec.dtype),
        ],
        scratch_shapes=[pltpu.VMEM((1, f), jnp.float32)],
        compiler_params=pltpu.CompilerParams(
            dimension_semantics=("parallel", "arbitrary"),
        ),
    )(a_mat, state_in, feat_spec)
    return out, final_state


# speedup vs baseline: 9.3479x; 1.0705x over previous
import functools

import jax
import jax.numpy as jnp
import numpy as np
from jax.experimental import pallas as pl
from jax.experimental.pallas import tpu as pltpu

_ALPHA = 0.99
_L = 200


def _decay_matrix(l: int) -> np.ndarray:
    i = np.arange(l, dtype=np.float64)
    p = i[:, None] - i[None, :]
    mat = np.tril((1.0 - _ALPHA) * np.power(_ALPHA, np.maximum(p, 0.0)))
    return mat.astype(np.float32)


def _carry_powers(l: int, f: int) -> np.ndarray:
    d = np.power(_ALPHA, np.arange(1, l + 1, dtype=np.float64))
    return np.broadcast_to(d[:, None], (l, f)).astype(np.float32)


def _ema_body(a_mat_ref, d_ref, state_ref, x_ref, out_ref, fs_ref, carry_ref, *,
              n_chunks: int, l: int):
    t = pl.program_id(1)

    @pl.when(t == 0)
    def _():
        carry_ref[...] = state_ref[0]

    x = x_ref[0]
    a = x[0] * x[0] + x[1] * x[1]

    s_prev = carry_ref[...]

    s_all = jnp.dot(a_mat_ref[...], a.astype(jnp.bfloat16),
                    preferred_element_type=jnp.float32)
    s_all = s_all + d_ref[...] * s_prev

    inv = jax.lax.rsqrt(s_all)
    out_ref[0, 0] = x[0] * inv
    out_ref[0, 1] = x[1] * inv

    carry_ref[...] = s_all[l - 1:l, :]

    @pl.when(t == n_chunks - 1)
    def _():
        fs_ref[0] = s_all[l - 1:l, :]


def kernel(feat_spec, state):
    b, c, t, f = feat_spec.shape
    l = _L
    n_chunks = t // l
    a_mat = jnp.asarray(_decay_matrix(l)).astype(jnp.bfloat16)
    d_pow = jnp.asarray(_carry_powers(l, f))
    state_in = state.astype(feat_spec.dtype)

    body = functools.partial(_ema_body, n_chunks=n_chunks, l=l)
    out, final_state = pl.pallas_call(
        body,
        grid=(b, n_chunks),
        in_specs=[
            pl.BlockSpec((l, l), lambda i, j: (0, 0)),
            pl.BlockSpec((l, f), lambda i, j: (0, 0)),
            pl.BlockSpec((1, 1, f), lambda i, j: (0, 0, 0)),
            pl.BlockSpec((1, c, l, f), lambda i, j: (i, 0, j, 0)),
        ],
        out_specs=[
            pl.BlockSpec((1, c, l, f), lambda i, j: (i, 0, j, 0)),
            pl.BlockSpec((1, 1, f), lambda i, j: (i, 0, 0)),
        ],
        out_shape=[
            jax.ShapeDtypeStruct((b, c, t, f), feat_spec.dtype),
            jax.ShapeDtypeStruct((b, 1, f), feat_spec.dtype),
        ],
        scratch_shapes=[pltpu.VMEM((1, f), jnp.float32)],
        compiler_params=pltpu.CompilerParams(
            dimension_semantics=("parallel", "arbitrary"),
        ),
    )(a_mat, d_pow, state_in, feat_spec)
    return out, final_state


# speedup vs baseline: 17.3254x; 1.8534x over previous
import functools

import jax
import jax.numpy as jnp
import numpy as np
from jax.experimental import pallas as pl
from jax.experimental.pallas import tpu as pltpu

_ALPHA = 0.99
_L = 200


def _decay_matrix(l: int) -> np.ndarray:
    i = np.arange(l, dtype=np.float64)
    p = i[:, None] - i[None, :]
    mat = np.tril((1.0 - _ALPHA) * np.power(_ALPHA, np.maximum(p, 0.0)))
    return mat.astype(np.float32)


def _carry_powers(l: int, f: int) -> np.ndarray:
    d = np.power(_ALPHA, np.arange(1, l + 1, dtype=np.float64))
    return np.broadcast_to(d[:, None], (l, f)).astype(np.float32)


def _ema_body(a_mat_ref, d_ref, state_ref, x_ref, out_ref, fs_ref, *,
              n_chunks: int, l: int):
    a_mat = a_mat_ref[...]
    d_pow = d_ref[...]

    def body(k, s_prev):
        base = pl.multiple_of(k * l, l)
        x0 = x_ref[0, 0, pl.ds(base, l), :]
        x1 = x_ref[0, 1, pl.ds(base, l), :]
        a = x0 * x0 + x1 * x1
        s_all = jnp.dot(a_mat, a.astype(jnp.bfloat16),
                        preferred_element_type=jnp.float32)
        s_all = s_all + d_pow * s_prev
        inv = jax.lax.rsqrt(s_all)
        out_ref[0, 0, pl.ds(base, l), :] = x0 * inv
        out_ref[0, 1, pl.ds(base, l), :] = x1 * inv
        return s_all[l - 1:l, :]

    s_fin = jax.lax.fori_loop(0, n_chunks, body, state_ref[0])
    fs_ref[0] = s_fin


def kernel(feat_spec, state):
    b, c, t, f = feat_spec.shape
    l = _L
    n_chunks = t // l
    a_mat = jnp.asarray(_decay_matrix(l)).astype(jnp.bfloat16)
    d_pow = jnp.asarray(_carry_powers(l, f))
    state_in = state.astype(feat_spec.dtype)

    body = functools.partial(_ema_body, n_chunks=n_chunks, l=l)
    out, final_state = pl.pallas_call(
        body,
        grid=(b,),
        in_specs=[
            pl.BlockSpec((l, l), lambda i: (0, 0)),
            pl.BlockSpec((l, f), lambda i: (0, 0)),
            pl.BlockSpec((1, 1, f), lambda i: (0, 0, 0)),
            pl.BlockSpec((1, c, t, f), lambda i: (i, 0, 0, 0)),
        ],
        out_specs=[
            pl.BlockSpec((1, c, t, f), lambda i: (i, 0, 0, 0)),
            pl.BlockSpec((1, 1, f), lambda i: (i, 0, 0)),
        ],
        out_shape=[
            jax.ShapeDtypeStruct((b, c, t, f), feat_spec.dtype),
            jax.ShapeDtypeStruct((b, 1, f), feat_spec.dtype),
        ],
        compiler_params=pltpu.CompilerParams(
            dimension_semantics=("parallel",),
            vmem_limit_bytes=56 * 1024 * 1024,
        ),
    )(a_mat, d_pow, state_in, feat_spec)
    return out, final_state


# speedup vs baseline: 19.7115x; 1.1377x over previous
import functools

import jax
import jax.numpy as jnp
import numpy as np
from jax.experimental import pallas as pl
from jax.experimental.pallas import tpu as pltpu

_ALPHA = 0.99
_L = 200


def _decay_matrix(l: int) -> np.ndarray:
    i = np.arange(l, dtype=np.float64)
    p = i[:, None] - i[None, :]
    mat = np.tril((1.0 - _ALPHA) * np.power(_ALPHA, np.maximum(p, 0.0)))
    return mat.astype(np.float32)


def _carry_powers(l: int, f: int) -> np.ndarray:
    d = np.power(_ALPHA, np.arange(1, l + 1, dtype=np.float64))
    return np.broadcast_to(d[:, None], (l, f)).astype(np.float32)


def _ema_body(a_mat_ref, d_ref, state_ref, x_ref, out_ref, fs_ref, *,
              n_chunks: int, l: int):
    a_mat = a_mat_ref[...]
    d_pow = d_ref[...]

    s_prev = state_ref[0]
    for k in range(n_chunks):
        base = k * l
        x0 = x_ref[0, 0, base:base + l, :]
        x1 = x_ref[0, 1, base:base + l, :]
        a = x0 * x0 + x1 * x1
        s_all = jnp.dot(a_mat, a.astype(jnp.bfloat16),
                        preferred_element_type=jnp.float32)
        s_all = s_all + d_pow * s_prev
        inv = jax.lax.rsqrt(s_all)
        out_ref[0, 0, base:base + l, :] = x0 * inv
        out_ref[0, 1, base:base + l, :] = x1 * inv
        s_prev = s_all[l - 1:l, :]
    fs_ref[0] = s_prev


def kernel(feat_spec, state):
    b, c, t, f = feat_spec.shape
    l = _L
    n_chunks = t // l
    a_mat = jnp.asarray(_decay_matrix(l)).astype(jnp.bfloat16)
    d_pow = jnp.asarray(_carry_powers(l, f))
    state_in = state.astype(feat_spec.dtype)

    body = functools.partial(_ema_body, n_chunks=n_chunks, l=l)
    out, final_state = pl.pallas_call(
        body,
        grid=(b,),
        in_specs=[
            pl.BlockSpec((l, l), lambda i: (0, 0)),
            pl.BlockSpec((l, f), lambda i: (0, 0)),
            pl.BlockSpec((1, 1, f), lambda i: (0, 0, 0)),
            pl.BlockSpec((1, c, t, f), lambda i: (i, 0, 0, 0)),
        ],
        out_specs=[
            pl.BlockSpec((1, c, t, f), lambda i: (i, 0, 0, 0)),
            pl.BlockSpec((1, 1, f), lambda i: (i, 0, 0)),
        ],
        out_shape=[
            jax.ShapeDtypeStruct((b, c, t, f), feat_spec.dtype),
            jax.ShapeDtypeStruct((b, 1, f), feat_spec.dtype),
        ],
        compiler_params=pltpu.CompilerParams(
            dimension_semantics=("parallel",),
            vmem_limit_bytes=56 * 1024 * 1024,
        ),
    )(a_mat, d_pow, state_in, feat_spec)
    return out, final_state
